# baseline (speedup 1.0000x reference)
import functools

import jax
import jax.numpy as jnp
from jax import lax
from jax.experimental import pallas as pl
from jax.experimental.pallas import tpu as pltpu
from jax.experimental.pallas import tpu_sc as plsc

B = 16384
L = 20
D = 128
TD = 3 * D
H = 1024
OUT = 1024

NC = 2
NS = 16
NW = NC * NS

CHUNK = 32
CL = CHUNK * L
NSPLIT = 4
SB = B // NSPLIT

_MESH = plsc.VectorSubcoreMesh(core_axis_name="c", subcore_axis_name="s")


def _make_pool(nb):
    spw = nb // NW
    nch = spw // CHUNK

    @functools.partial(
        pl.kernel,
        mesh=_MESH,
        out_type=jax.ShapeDtypeStruct((nb, TD), jnp.float32),
        scratch_types=[
            pltpu.VMEM((3 * spw * L,), jnp.int32),
            pltpu.VMEM((CL, D), jnp.float32),
            pltpu.VMEM((CHUNK, TD), jnp.float32),
            pltpu.SemaphoreType.DMA((2,)),
        ],
    )
    def _pool(xw, x2, x3, tw, t2, t3, out, idx_v, rows_v, acc_v, sems):
        wid = lax.axis_index("s") * NC + lax.axis_index("c")
        base = wid * spw

        for t, xh in enumerate((xw, x2, x3)):
            pltpu.sync_copy(xh.at[pl.ds(wid * spw * L, spw * L)],
                            idx_v.at[pl.ds(t * spw * L, spw * L)])

        def chunk_body(c, carry):
            s0 = base + c * CHUNK

            for t, th in enumerate((tw, t2, t3)):
                pltpu.async_copy(
                    th.at[idx_v.at[pl.ds((t * nch + c) * CL, CL)]],
                    rows_v, sems.at[0])

                def samp_body(s, carry2):
                    r = s * L
                    for v in range(D // 16):
                        col = pl.ds(v * 16, 16)
                        accv = rows_v[r, col]
                        for l in range(1, L):
                            accv = accv + rows_v[r + l, col]
                        acc_v[s, pl.ds(t * D + v * 16, 16)] = (
                            accv * (1.0 / L))
                    return carry2

                pltpu.make_async_copy(th.at[pl.ds(0, CL)], rows_v,
                                      sems.at[0]).wait()
                lax.fori_loop(0, CHUNK, samp_body, 0)

            pltpu.sync_copy(acc_v, out.at[pl.ds(s0, CHUNK)])
            return carry

        lax.fori_loop(0, nch, chunk_body, 0)

    return _pool


BM = 512


def _mlp_body(p_ref, w1_ref, b1_ref, w2_ref, b2_ref, o_ref):
    h = jnp.dot(p_ref[...], w1_ref[...], preferred_element_type=jnp.float32)
    h = jnp.maximum(h + b1_ref[...], 0.0)
    o_ref[...] = (
        jnp.dot(h, w2_ref[...], preferred_element_type=jnp.float32)
        + b2_ref[...]
    )


def _make_mlp(nb):
    return pl.pallas_call(
        _mlp_body,
        grid=(nb // BM,),
        in_specs=[
            pl.BlockSpec((BM, TD), lambda i: (i, 0)),
            pl.BlockSpec((TD, H), lambda i: (0, 0)),
            pl.BlockSpec((1, H), lambda i: (0, 0)),
            pl.BlockSpec((H, OUT), lambda i: (0, 0)),
            pl.BlockSpec((1, OUT), lambda i: (0, 0)),
        ],
        out_specs=pl.BlockSpec((BM, OUT), lambda i: (i, 0)),
        out_shape=jax.ShapeDtypeStruct((nb, OUT), jnp.float32),
    )


_pool_sb = _make_pool(SB)
_mlp_sb = _make_mlp(SB)


def kernel(x, emb_word, emb_ngram2, emb_ngram3, W1, b1, W2, b2):
    b1r = b1.reshape(1, H)
    b2r = b2.reshape(1, OUT)
    outs = []
    for i in range(NSPLIT):
        sl = slice(i * SB, (i + 1) * SB)
        pooled = _pool_sb(x[0, sl].reshape(SB * L), x[2, sl].reshape(SB * L),
                          x[3, sl].reshape(SB * L),
                          emb_word, emb_ngram2, emb_ngram3)
        outs.append(_mlp_sb(pooled, W1, b1r, W2, b2r))
    return jnp.concatenate(outs, axis=0)
